# Initial kernel scaffold; baseline (speedup 1.0000x reference)
import jax
import jax.numpy as jnp
from jax.experimental import pallas as pl


def kernel(x, edge_index, edge_attr, params):
    raise NotImplementedError("write your pallas kernel here")


# speedup vs baseline: 1.6949x; 1.6949x over previous
import functools

import jax
import jax.numpy as jnp
from jax import lax
from jax.experimental import pallas as pl
from jax.experimental.pallas import tpu as pltpu
from jax.experimental.pallas import tpu_sc as plsc

N = 10000
D = 128
H = 64
ED = 4
L = 4
OUT = 4

NPAD = 10240
W = 128
CHUNK = 128
NW = 32


def _ln(x):
    m = jnp.mean(x, axis=-1, keepdims=True)
    v = jnp.mean((x - m) ** 2, axis=-1, keepdims=True)
    return (x - m) / jnp.sqrt(v + 1e-5)


def _act(x, kind):
    if kind == 'gelu':
        return 0.5 * x * (1.0 + lax.erf(x * 0.7071067811865476))
    return jnp.where(x >= 0, x, 0.1 * x)


def _dot(a, b):
    return jnp.dot(a, b, preferred_element_type=jnp.float32)


def _padrows(w):
    return jnp.concatenate([w, jnp.zeros((W - w.shape[0], w.shape[1]),
                                         jnp.float32)], axis=0)


def _embed_body(x_ref, wl_ref, bl_ref, wp_ref, bp_ref, wc1_ref, wc2_ref,
                bc_ref, o_ref):
    x = x_ref[...]
    lin = _dot(x, wl_ref[...]) + bl_ref[...]
    pw = _dot(x * x, wp_ref[...]) + bp_ref[...]
    h = _dot(lin, wc1_ref[...]) + _dot(pw, wc2_ref[...]) + bc_ref[...]
    o_ref[...] = jnp.concatenate([h, jnp.zeros_like(h)], axis=-1)


def _embed(xp, wl, bl, wp, bp, wc1, wc2, bc):
    nb = NPAD // 1024
    full = lambda a: pl.BlockSpec(a.shape, lambda i: (0,) * a.ndim)
    return pl.pallas_call(
        _embed_body,
        grid=(nb,),
        in_specs=[pl.BlockSpec((1024, D), lambda i: (i, 0)),
                  full(wl), full(bl), full(wp), full(bp),
                  full(wc1), full(wc2), full(bc)],
        out_specs=pl.BlockSpec((1024, W), lambda i: (i, 0)),
        out_shape=jax.ShapeDtypeStruct((NPAD, W), jnp.float32),
    )(xp, wl, bl, wp, bp, wc1, wc2, bc)


def _make_selfmask(epad):
    per_tile = epad // NW
    nchunk = per_tile // CHUNK
    mesh = plsc.VectorSubcoreMesh(core_axis_name="c", subcore_axis_name="s")

    @functools.partial(
        pl.kernel, mesh=mesh,
        out_type=jax.ShapeDtypeStruct((epad,), jnp.float32),
        scratch_types=[pltpu.VMEM((CHUNK,), jnp.int32),
                       pltpu.VMEM((CHUNK,), jnp.int32),
                       pltpu.VMEM((CHUNK,), jnp.float32)],
    )
    def k(src_hbm, dst_hbm, out_hbm, sv, dv, mv):
        wid = lax.axis_index("s") * 2 + lax.axis_index("c")
        base = wid * per_tile

        def body(j, carry):
            off = base + j * CHUNK
            pltpu.sync_copy(src_hbm.at[pl.ds(off, CHUNK)], sv)
            pltpu.sync_copy(dst_hbm.at[pl.ds(off, CHUNK)], dv)
            for q in range(CHUNK // 16):
                s16 = sv[pl.ds(q * 16, 16)]
                d16 = dv[pl.ds(q * 16, 16)]
                mv[pl.ds(q * 16, 16)] = jnp.where(
                    s16 == d16, jnp.float32(1.0), jnp.float32(0.0))
            pltpu.sync_copy(mv, out_hbm.at[pl.ds(off, CHUNK)])
            return carry

        lax.fori_loop(0, nchunk, body, 0)

    return k


def _make_gather(epad):
    per_tile = epad // NW
    nchunk = per_tile // CHUNK
    mesh = plsc.VectorSubcoreMesh(core_axis_name="c", subcore_axis_name="s")

    @functools.partial(
        pl.kernel, mesh=mesh,
        out_type=[jax.ShapeDtypeStruct((epad, W), jnp.float32),
                  jax.ShapeDtypeStruct((epad, W), jnp.float32)],
        scratch_types=[pltpu.VMEM((CHUNK,), jnp.int32),
                       pltpu.VMEM((CHUNK,), jnp.int32),
                       pltpu.VMEM((CHUNK, W), jnp.float32),
                       pltpu.VMEM((CHUNK, W), jnp.float32),
                       pltpu.SemaphoreType.DMA,
                       pltpu.SemaphoreType.DMA],
    )
    def k(h_hbm, src_hbm, dst_hbm, gs_hbm, gd_hbm,
          si, di, rs, rd, sem1, sem2):
        wid = lax.axis_index("s") * 2 + lax.axis_index("c")
        base = wid * per_tile

        def body(j, carry):
            off = base + j * CHUNK
            pltpu.sync_copy(src_hbm.at[pl.ds(off, CHUNK)], si)
            pltpu.sync_copy(dst_hbm.at[pl.ds(off, CHUNK)], di)
            c1 = pltpu.async_copy(h_hbm.at[si], rs, sem1)
            c2 = pltpu.async_copy(h_hbm.at[di], rd, sem2)
            c1.wait()
            c2.wait()
            pltpu.sync_copy(rs, gs_hbm.at[pl.ds(off, CHUNK)])
            pltpu.sync_copy(rd, gd_hbm.at[pl.ds(off, CHUNK)])
            return carry

        lax.fori_loop(0, nchunk, body, 0)

    return k


NHALF = NPAD // 2
ACC_ROWS = NHALF + 8


def _make_scatter(epad):
    per_tile = epad // 16
    nchunk = per_tile // CHUNK
    rows_per_tile = NHALF // 16
    mesh = plsc.VectorSubcoreMesh(core_axis_name="c", subcore_axis_name="s")

    @functools.partial(
        pl.kernel, mesh=mesh,
        out_type=jax.ShapeDtypeStruct((NPAD, W), jnp.float32),
        scratch_types=[pltpu.VMEM((CHUNK,), jnp.int32),
                       pltpu.VMEM((CHUNK,), jnp.int32),
                       pltpu.VMEM((CHUNK, W), jnp.float32),
                       pltpu.VMEM((rows_per_tile, W), jnp.float32),
                       pltpu.VMEM_SHARED((ACC_ROWS, W), jnp.float32)],
    )
    def k(msg_hbm, dst_hbm, zero_hbm, out_hbm, iv, lv, rv, wb, accum):
        c = lax.axis_index("c")
        s = lax.axis_index("s")
        lo = c * NHALF

        @pl.when(s == 0)
        def _():
            pltpu.sync_copy(zero_hbm, accum)

        plsc.subcore_barrier()
        base = s * per_tile

        def body(j, carry):
            off = base + j * CHUNK
            pltpu.sync_copy(dst_hbm.at[pl.ds(off, CHUNK)], iv)
            pltpu.sync_copy(msg_hbm.at[pl.ds(off, CHUNK)], rv)
            for q in range(CHUNK // 16):
                d16 = iv[pl.ds(q * 16, 16)]
                loc = d16 - lo
                ok = (loc >= 0) & (loc < NHALF)
                lv[pl.ds(q * 16, 16)] = jnp.where(ok, loc, NHALF)
            pltpu.sync_copy(rv, accum.at[lv], add=True)
            return carry

        lax.fori_loop(0, nchunk, body, 0)
        plsc.subcore_barrier()
        r0 = s * rows_per_tile
        pltpu.sync_copy(accum.at[pl.ds(r0, rows_per_tile)], wb)
        pltpu.sync_copy(wb, out_hbm.at[pl.ds(lo + r0, rows_per_tile)])

    return k


def _edge_body(kind, gd_ref, gs_ref, ea_ref, mk_ref,
               w1d, w1s, w1e, b1, w21, b21,
               w2d, w2s, w2e, b2, w22, b22,
               a1, a2, ab, sc_ref, o_ref):
    gd = gd_ref[...]
    gs = gs_ref[...]
    ea = ea_ref[...]
    m1p = (_dot(gd, w1d[...]) + _dot(gs, w1s[...]) + _dot(ea, w1e[...])
           + b1[...])
    m1 = _dot(_ln(_act(m1p, kind)), w21[...]) + b21[...]
    m2p = (_dot(gd, w2d[...]) + _dot(gs, w2s[...]) + _dot(ea, w2e[...])
           + b2[...])
    m2 = _dot(_ln(_act(m2p, 'gelu')), w22[...]) + b22[...]
    z = _dot(m1, a1[...]) + _dot(m2, a2[...]) + ab[...]
    z = z - jnp.max(z, axis=-1, keepdims=True)
    e = jnp.exp(z)
    aw = e / jnp.sum(e, axis=-1, keepdims=True)
    msg = aw[:, 0:1] * m1 + aw[:, 1:2] * m2
    mk = mk_ref[...]
    scale = mk + (1.0 - mk) * sc_ref[0, 0]
    msg = msg * scale
    o_ref[...] = jnp.concatenate([msg, jnp.zeros_like(msg)], axis=-1)


def _edge(kind, epad, gd, gs, ea, mk, weights):
    eb = 2048
    nb = epad // eb
    full = lambda a: pl.BlockSpec(a.shape, lambda i: (0,) * a.ndim)
    return pl.pallas_call(
        functools.partial(_edge_body, kind),
        grid=(nb,),
        in_specs=[pl.BlockSpec((eb, W), lambda i: (i, 0)),
                  pl.BlockSpec((eb, W), lambda i: (i, 0)),
                  pl.BlockSpec((eb, ED), lambda i: (i, 0)),
                  pl.BlockSpec((eb, 1), lambda i: (i, 0))]
                 + [full(w) for w in weights],
        out_specs=pl.BlockSpec((eb, W), lambda i: (i, 0)),
        out_shape=jax.ShapeDtypeStruct((epad, W), jnp.float32),
    )(gd, gs, ea, mk, *weights)


def _update_body(kind, residual, ag_ref, h_ref, acc_ref,
                 wga, wgh, bg, w1a, w1h, b1u, w2, b2u, at_ref,
                 ho_ref, ao_ref):
    aggr = ag_ref[...]
    hw = h_ref[...]
    h = hw[:, :H]
    gate = jax.nn.sigmoid(_dot(aggr, wga[...]) + _dot(hw, wgh[...]) + bg[...])
    u1 = _ln(_act(_dot(aggr, w1a[...]) + _dot(hw, w1h[...]) + b1u[...], kind))
    u2 = h + _act(_dot(u1, w2[...]) + b2u[...], kind)
    hn = _ln(h * (1.0 - gate) + u2 * gate)
    ho = h + hn if residual else hn
    ho_ref[...] = jnp.concatenate([ho, jnp.zeros_like(ho)], axis=-1)
    ao_ref[...] = acc_ref[...] + at_ref[0, 0] * ho


def _update(kind, residual, ag, h, acc, weights):
    nb = NPAD // 1024
    full = lambda a: pl.BlockSpec(a.shape, lambda i: (0,) * a.ndim)
    return pl.pallas_call(
        functools.partial(_update_body, kind, residual),
        grid=(nb,),
        in_specs=[pl.BlockSpec((1024, W), lambda i: (i, 0)),
                  pl.BlockSpec((1024, W), lambda i: (i, 0)),
                  pl.BlockSpec((1024, H), lambda i: (i, 0))]
                 + [full(w) for w in weights],
        out_specs=[pl.BlockSpec((1024, W), lambda i: (i, 0)),
                   pl.BlockSpec((1024, H), lambda i: (i, 0))],
        out_shape=[jax.ShapeDtypeStruct((NPAD, W), jnp.float32),
                   jax.ShapeDtypeStruct((NPAD, H), jnp.float32)],
    )(ag, h, acc, *weights)


def _readout_body(h_ref, p1, pb1, p2, pb2, ow, ob, o_ref):
    rows = lax.broadcasted_iota(jnp.int32, (NPAD, 1), 0)
    hm = jnp.where(rows < N, h_ref[...], 0.0)
    g = jnp.sum(hm, axis=0, keepdims=True) * (1.0 / N)
    g = _ln(_act(_dot(g, p1[...]) + pb1[...], 'leaky'))
    g = _act(_dot(g, p2[...]) + pb2[...], 'leaky')
    o_ref[...] = _dot(g, ow[...]) + ob[...]


def _readout(hacc, p1, pb1, p2, pb2, ow, ob):
    args = (hacc, p1, pb1, p2, pb2, ow, ob)
    return pl.pallas_call(
        _readout_body,
        out_shape=jax.ShapeDtypeStruct((1, OUT), jnp.float32),
    )(*args)


def kernel(x, edge_index, edge_attr, params):
    p = params
    E = edge_index.shape[1]
    etot = E + N
    epad = -(-etot // (NW * CHUNK)) * (NW * CHUNK)

    xp = jnp.zeros((NPAD, D), jnp.float32).at[:N].set(x)
    sl = jnp.arange(N, dtype=edge_index.dtype)
    src = jnp.concatenate([edge_index[0], sl,
                           jnp.zeros((epad - etot,), edge_index.dtype)])
    dst = jnp.concatenate([edge_index[1], sl,
                           jnp.zeros((epad - etot,), edge_index.dtype)])
    src = src.astype(jnp.int32)
    dstg = dst.astype(jnp.int32)
    dsts = jnp.concatenate(
        [dstg[:etot], jnp.full((epad - etot,), N, jnp.int32)])
    dummy = jnp.zeros((N, ED), jnp.float32).at[:, 0].set(1.0)
    ea = jnp.concatenate([edge_attr, dummy,
                          jnp.zeros((epad - etot, ED), jnp.float32)])
    zeros_acc = jnp.zeros((ACC_ROWS, W), jnp.float32)

    r1 = lambda b: b.reshape(1, -1)
    wl = p['emb_lin_w'].T
    wpw = p['emb_pow_w'].T
    wcT = p['emb_comb_w'].T
    h = _embed(xp, wl, r1(p['emb_lin_b']), wpw, r1(p['emb_pow_b']),
               wcT[:H], wcT[H:], r1(p['emb_comb_b']))

    mask_k = _make_selfmask(epad)
    gather_k = _make_gather(epad)
    scatter_k = _make_scatter(epad)

    mk = mask_k(src, dstg).reshape(epad, 1)

    attn = jax.nn.softmax(p['layer_attn'])
    acc = jnp.zeros((NPAD, H), jnp.float32)
    for i in range(L):
        kind = 'gelu' if i % 2 == 1 else 'leaky'
        gs, gd = gather_k(h, src, dstg)
        w1T = p['mp1_w1'][i].T
        w2T = p['mp2_w1'][i].T
        aT = p['attn_w'][i].T
        ew = [_padrows(w1T[:H]), _padrows(w1T[H:2 * H]), w1T[2 * H:],
              r1(p['mp1_b1'][i]),
              p['mp1_w2'][i].T, r1(p['mp1_b2'][i]),
              _padrows(w2T[:H]), _padrows(w2T[H:2 * H]), w2T[2 * H:],
              r1(p['mp2_b1'][i]),
              p['mp2_w2'][i].T, r1(p['mp2_b2'][i]),
              aT[:H], aT[H:], r1(p['attn_b'][i]),
              p['scale_factor'][i].reshape(1, 1)]
        msg = _edge(kind, epad, gd, gs, ea, mk, ew)
        ag = scatter_k(msg, dsts, zeros_acc)
        wgT = p['gate_w'][i].T
        w1uT = p['upd1_w'][i].T
        uw = [_padrows(wgT[:H]), _padrows(wgT[H:]), r1(p['gate_b'][i]),
              _padrows(w1uT[:H]), _padrows(w1uT[H:]), r1(p['upd1_b'][i]),
              p['upd2_w'][i].T, r1(p['upd2_b'][i]),
              attn[i].reshape(1, 1)]
        h, acc = _update(kind, i % 2 == 1, ag, h, acc, uw)

    return _readout(acc, p['pre_w1'].T, r1(p['pre_b1']),
                    p['pre_w2'].T, r1(p['pre_b2']),
                    p['out_w'].T, r1(p['out_b']))
